# baseline (speedup 1.0000x reference)
import jax
import jax.numpy as jnp
from jax.experimental import pallas as pl

_B, _P, _D = 64, 1024, 768
_BPB = 1


def _tc_body(x_ref, p_ref, o_ref):
    for b in range(_BPB):
        sl = slice(b * _P, (b + 1) * _P)
        o_ref[sl, :] = x_ref[sl, :] + p_ref[...]


def kernel(encoded_patches, pos_table):
    x2 = encoded_patches.reshape(_B * _P, _D)
    out = pl.pallas_call(
        _tc_body,
        grid=(_B // _BPB,),
        in_specs=[
            pl.BlockSpec((_BPB * _P, _D), lambda i: (i, 0)),
            pl.BlockSpec((_P, _D), lambda i: (0, 0)),
        ],
        out_specs=pl.BlockSpec((_BPB * _P, _D), lambda i: (i, 0)),
        out_shape=jax.ShapeDtypeStruct((_B * _P, _D), jnp.float32),
    )(x2, pos_table)
    return out.reshape(_B, _P, _D)


# speedup vs baseline: 1.0422x; 1.0422x over previous
import jax
import jax.numpy as jnp
from jax.experimental import pallas as pl

_B, _P, _D = 64, 1024, 768
_BPB = 4


def _tc_body(x_ref, p_ref, o_ref):
    for b in range(_BPB):
        sl = slice(b * _P, (b + 1) * _P)
        o_ref[sl, :] = x_ref[sl, :] + p_ref[...]


def kernel(encoded_patches, pos_table):
    x2 = encoded_patches.reshape(_B * _P, _D)
    out = pl.pallas_call(
        _tc_body,
        grid=(_B // _BPB,),
        in_specs=[
            pl.BlockSpec((_BPB * _P, _D), lambda i: (i, 0)),
            pl.BlockSpec((_P, _D), lambda i: (0, 0)),
        ],
        out_specs=pl.BlockSpec((_BPB * _P, _D), lambda i: (i, 0)),
        out_shape=jax.ShapeDtypeStruct((_B * _P, _D), jnp.float32),
    )(x2, pos_table)
    return out.reshape(_B, _P, _D)
